# baseline (speedup 1.0000x reference)
import functools

import jax
import jax.numpy as jnp
from jax import lax
from jax.experimental import pallas as pl
from jax.experimental.pallas import tpu as pltpu
from jax.experimental.pallas import tpu_sc as plsc

VOCAB = 256
DIM = 64
NUM_CORES = 2
NUM_SUBCORES = 16
NW = NUM_CORES * NUM_SUBCORES


def _table_body(embed_ref, w_ref, b_ref, m_ref):
    m = lax.dot_general(
        embed_ref[...], w_ref[...],
        (((1,), (1,)), ((), ())),
        preferred_element_type=jnp.float32,
    )
    m_ref[...] = m + b_ref[...]


def _build_table(embed, W, b2d):
    return pl.pallas_call(
        _table_body,
        out_shape=jax.ShapeDtypeStruct((VOCAB, VOCAB), jnp.float32),
    )(embed, W, b2d)


def _make_gather(total_tokens: int):
    b_per_w = total_tokens // NW
    chunk = 128
    nbuf = 3
    n_chunks = b_per_w // chunk
    mesh = plsc.VectorSubcoreMesh(
        core_axis_name="c", subcore_axis_name="s",
        num_cores=NUM_CORES, num_subcores=NUM_SUBCORES)

    @functools.partial(
        pl.kernel,
        out_type=jax.ShapeDtypeStruct((total_tokens, VOCAB), jnp.float32),
        mesh=mesh,
        scratch_types=[
            pltpu.VMEM((b_per_w,), jnp.int32),
        ] + [pltpu.VMEM((chunk, VOCAB), jnp.float32) for _ in range(nbuf)]
          + [pltpu.SemaphoreType.DMA for _ in range(nbuf)]
          + [pltpu.SemaphoreType.DMA for _ in range(nbuf)],
    )
    def gather(table_hbm, idx_hbm, out_hbm, idx_all, *bufs_and_sems):
        row_bufs = bufs_and_sems[:nbuf]
        sems_in = bufs_and_sems[nbuf:2 * nbuf]
        sems_out = bufs_and_sems[2 * nbuf:]
        wid = lax.axis_index("s") * NUM_CORES + lax.axis_index("c")
        base = wid * b_per_w

        pltpu.sync_copy(idx_hbm.at[pl.ds(base, b_per_w)], idx_all)

        def gather_copy(c):
            slot = c % nbuf
            return pltpu.make_async_copy(
                table_hbm.at[idx_all.at[pl.ds(c * chunk, chunk)]],
                row_bufs[slot], sems_in[slot])

        def scatter_copy(c):
            slot = c % nbuf
            return pltpu.make_async_copy(
                row_bufs[slot], out_hbm.at[pl.ds(base + c * chunk, chunk)],
                sems_out[slot])

        for c in range(min(nbuf, n_chunks)):
            gather_copy(c).start()
        for c in range(n_chunks):
            gather_copy(c).wait()
            scatter_copy(c).start()
            p = c - 1 + nbuf
            if c >= 1 and p < n_chunks:
                scatter_copy(c - 1).wait()
                gather_copy(p).start()
        for c in range(max(0, n_chunks - nbuf), n_chunks):
            scatter_copy(c).wait()

    return gather


def _onehot_body(x_ref, m_ref, out_ref):
    xb = x_ref[0, 0, :]
    oh = (xb[:, None] == lax.broadcasted_iota(jnp.int32, (1, VOCAB), 1)
          ).astype(jnp.float32)
    out_ref[...] = lax.dot_general(
        oh, m_ref[...], (((1,), (0,)), ((), ())),
        preferred_element_type=jnp.float32)


def _onehot_lookup(table, flat, total):
    blk = 1024
    nb = total // blk
    x3 = flat.reshape(nb, 1, blk)
    return pl.pallas_call(
        _onehot_body,
        grid=(nb,),
        in_specs=[
            pl.BlockSpec((1, 1, blk), lambda i: (i, 0, 0)),
            pl.BlockSpec((VOCAB, VOCAB), lambda i: (0, 0)),
        ],
        out_specs=pl.BlockSpec((blk, VOCAB), lambda i: (i, 0)),
        out_shape=jax.ShapeDtypeStruct((total, VOCAB), jnp.float32),
    )(x3, table)


def kernel(x, embed, W, b):
    B, S = x.shape
    total = B * S
    table = _build_table(embed, W, b.reshape(1, VOCAB))
    flat = x.reshape(total)
    out = _make_gather(total)(table, flat)
    return out.reshape(B, S, VOCAB)
